# baseline (speedup 1.0000x reference)
import jax
import jax.numpy as jnp
from jax import lax
from jax.experimental import pallas as pl
from jax.experimental.pallas import tpu as pltpu
from jax.experimental.pallas import tpu_sc as plsc

_NC = 2
_NS = 16
_L = 16
_NW = _NC * _NS
_B = 16384
_BPW = _B // _NW
_CHUNK = 128
_NCHUNK = _BPW // _CHUNK
_D = 32


def _mf_body(user_hbm, item_hbm, u_hbm, v_hbm, out_hbm,
             idx_u, idx_v, rows_u, rows_v, out_buf, sem):
    wid = lax.axis_index("s") * _NC + lax.axis_index("c")

    pltpu.sync_copy(user_hbm.at[wid], idx_u)
    pltpu.sync_copy(item_hbm.at[wid], idx_v)

    copies = []
    for c in range(_NCHUNK):
        copies.append(pltpu.async_copy(
            u_hbm.at[idx_u.at[c]], rows_u.at[pl.ds(c * _CHUNK, _CHUNK)], sem))
        copies.append(pltpu.async_copy(
            v_hbm.at[idx_v.at[c]], rows_v.at[pl.ds(c * _CHUNK, _CHUNK)], sem))
    for cp in copies:
        cp.wait()

    iota = lax.iota(jnp.int32, _L)

    def body(g, carry):
        b_idx = g * _L + iota
        acc = jnp.zeros((_L,), jnp.float32)
        for j in range(_D):
            jv = jnp.full((_L,), j, jnp.int32)
            uu = plsc.load_gather(rows_u, [b_idx, jv])
            vv = plsc.load_gather(rows_v, [b_idx, jv])
            acc = acc + uu * vv
        out_buf[pl.ds(g * _L, _L)] = acc
        return carry

    lax.fori_loop(0, _BPW // _L, body, 0)

    pltpu.sync_copy(out_buf, out_hbm.at[pl.ds(wid * _BPW, _BPW)])


def kernel(user, item, U, V):
    user3 = user.reshape(_NW, _NCHUNK, _CHUNK)
    item3 = item.reshape(_NW, _NCHUNK, _CHUNK)
    mesh = plsc.VectorSubcoreMesh(core_axis_name="c", subcore_axis_name="s")
    fn = pl.kernel(
        _mf_body,
        mesh=mesh,
        out_type=jax.ShapeDtypeStruct((_B,), jnp.float32),
        compiler_params=pltpu.CompilerParams(
            needs_layout_passes=False, use_tc_tiling_on_sc=False),
        scratch_types=[
            pltpu.VMEM((_NCHUNK, _CHUNK), jnp.int32),
            pltpu.VMEM((_NCHUNK, _CHUNK), jnp.int32),
            pltpu.VMEM((_BPW, _D), jnp.float32),
            pltpu.VMEM((_BPW, _D), jnp.float32),
            pltpu.VMEM((_BPW,), jnp.float32),
            pltpu.SemaphoreType.DMA,
        ],
    )
    return fn(user3, item3, U, V)
